# Initial kernel scaffold; baseline (speedup 1.0000x reference)
import jax
import jax.numpy as jnp
from jax.experimental import pallas as pl


def kernel(input_ids, attention_mask, embed_weight):
    raise NotImplementedError("write your pallas kernel here")


# speedup vs baseline: 2.3585x; 2.3585x over previous
import functools

import jax
import jax.numpy as jnp
from jax import lax
from jax.experimental import pallas as pl
from jax.experimental.pallas import tpu as pltpu
from jax.experimental.pallas import tpu_sc as plsc


_INFO = plsc.get_sparse_core_info()
_NC = _INFO.num_cores
_NS = _INFO.num_subcores
_NW = _NC * _NS
_CH = 128


@functools.partial(jax.jit, static_argnums=(2, 3))
def _sc_gather(idx3, table, nch, hidden):
    n_rows = _NW * nch * _CH
    mesh = plsc.VectorSubcoreMesh(core_axis_name="c", subcore_axis_name="s")

    @functools.partial(
        pl.kernel,
        out_type=jax.ShapeDtypeStruct((n_rows, hidden), jnp.float32),
        mesh=mesh,
        scratch_types=[
            pltpu.VMEM((nch, _CH), jnp.int32),
            pltpu.VMEM((_CH, hidden), jnp.float32),
            pltpu.VMEM((_CH, hidden), jnp.float32),
            pltpu.SemaphoreType.DMA,
            pltpu.SemaphoreType.DMA,
            pltpu.SemaphoreType.DMA,
            pltpu.SemaphoreType.DMA,
        ],
    )
    def body(idx_hbm, table_hbm, out_hbm, idx_v, rows_a, rows_b, gsem_a, gsem_b, ssem_a, ssem_b):
        wid = lax.axis_index("s") * _NC + lax.axis_index("c")
        base = wid * (nch * _CH)
        pltpu.sync_copy(idx_hbm.at[wid], idx_v)

        bufs = (rows_a, rows_b)
        gsems = (gsem_a, gsem_b)
        ssems = (ssem_a, ssem_b)

        g0 = pltpu.async_copy(table_hbm.at[idx_v.at[0]], bufs[0], gsems[0])
        gathers = [g0, None]
        stores = [None, None]
        for c in range(nch):
            b = c & 1
            nb = b ^ 1
            if c + 1 < nch:
                if stores[nb] is not None:
                    stores[nb].wait()
                    stores[nb] = None
                gathers[nb] = pltpu.async_copy(
                    table_hbm.at[idx_v.at[c + 1]], bufs[nb], gsems[nb]
                )
            gathers[b].wait()
            stores[b] = pltpu.async_copy(
                bufs[b], out_hbm.at[pl.ds(base + c * _CH, _CH)], ssems[b]
            )
        for s in stores:
            if s is not None:
                s.wait()

    return body(idx3, table)


def kernel(input_ids, attention_mask, embed_weight):
    del attention_mask
    batch, seq = input_ids.shape
    vocab, hidden = embed_weight.shape
    n_rows = batch * seq
    nch = n_rows // (_NW * _CH)
    ids = input_ids.reshape(-1).astype(jnp.int32)
    idx3 = ids.reshape(_NW, nch, _CH)
    table = embed_weight.astype(jnp.float32)
    out = _sc_gather(idx3, table, nch, hidden)
    return out.reshape(batch, seq, hidden)


# speedup vs baseline: 2.4679x; 1.0464x over previous
import functools

import jax
import jax.numpy as jnp
from jax import lax
from jax.experimental import pallas as pl
from jax.experimental.pallas import tpu as pltpu
from jax.experimental.pallas import tpu_sc as plsc


_INFO = plsc.get_sparse_core_info()
_NC = _INFO.num_cores
_NS = _INFO.num_subcores
_NW = _NC * _NS
_CH = 128


@functools.partial(jax.jit, static_argnums=(2, 3))
def _sc_gather(idx3, table, nch, hidden):
    n_rows = _NW * nch * _CH
    mesh = plsc.VectorSubcoreMesh(core_axis_name="c", subcore_axis_name="s")

    nbuf = 4
    @functools.partial(
        pl.kernel,
        out_type=jax.ShapeDtypeStruct((n_rows, hidden), jnp.float32),
        mesh=mesh,
        scratch_types=[
            pltpu.VMEM((nch, _CH), jnp.int32),
            [pltpu.VMEM((_CH, hidden), jnp.float32)] * nbuf,
            [pltpu.SemaphoreType.DMA] * nbuf,
            [pltpu.SemaphoreType.DMA] * nbuf,
        ],
    )
    def body(idx_hbm, table_hbm, out_hbm, idx_v, bufs, gsems, ssems):
        wid = lax.axis_index("s") * _NC + lax.axis_index("c")
        base = wid * (nch * _CH)
        pltpu.sync_copy(idx_hbm.at[wid], idx_v)

        gathers = [None] * nbuf
        stores = [None] * nbuf
        for c in range(min(nbuf, nch)):
            gathers[c] = pltpu.async_copy(table_hbm.at[idx_v.at[c]], bufs[c], gsems[c])
        for c in range(nch):
            b = c % nbuf
            gathers[b].wait()
            stores[b] = pltpu.async_copy(
                bufs[b], out_hbm.at[pl.ds(base + c * _CH, _CH)], ssems[b]
            )
            nxt = c + nbuf
            if nxt < nch:
                stores[b].wait()
                gathers[b] = pltpu.async_copy(
                    table_hbm.at[idx_v.at[nxt]], bufs[b], gsems[b]
                )
                stores[b] = None
        for s in stores:
            if s is not None:
                s.wait()

    return body(idx3, table)


def kernel(input_ids, attention_mask, embed_weight):
    del attention_mask
    batch, seq = input_ids.shape
    vocab, hidden = embed_weight.shape
    n_rows = batch * seq
    nch = n_rows // (_NW * _CH)
    ids = input_ids.reshape(-1).astype(jnp.int32)
    idx3 = ids.reshape(_NW, nch, _CH)
    table = embed_weight.astype(jnp.float32)
    out = _sc_gather(idx3, table, nch, hidden)
    return out.reshape(batch, seq, hidden)


# speedup vs baseline: 4.4079x; 1.7861x over previous
import functools

import jax
import jax.numpy as jnp
from jax import lax
from jax.experimental import pallas as pl
from jax.experimental.pallas import tpu as pltpu
from jax.experimental.pallas import tpu_sc as plsc


_INFO = plsc.get_sparse_core_info()
_NC = _INFO.num_cores
_NS = _INFO.num_subcores
_NW = _NC * _NS
_CH = 128


@functools.partial(jax.jit, static_argnums=(2, 3))
def _sc_gather(idx3, table, nch, hidden):
    n_rows = _NW * nch * _CH
    mesh = plsc.VectorSubcoreMesh(core_axis_name="c", subcore_axis_name="s")

    nbuf = 4
    vocab = table.shape[0]
    @functools.partial(
        pl.kernel,
        out_type=jax.ShapeDtypeStruct((n_rows, hidden), jnp.float32),
        mesh=mesh,
        scratch_types=[
            pltpu.VMEM((nch, _CH), jnp.int32),
            [pltpu.VMEM((_CH, hidden), jnp.float32)] * nbuf,
            pltpu.VMEM_SHARED((vocab, hidden), jnp.float32),
            [pltpu.SemaphoreType.DMA] * nbuf,
            [pltpu.SemaphoreType.DMA] * nbuf,
        ],
    )
    def body(idx_hbm, table_hbm, out_hbm, idx_v, bufs, tab_sh, gsems, ssems):
        wid = lax.axis_index("s") * _NC + lax.axis_index("c")
        base = wid * (nch * _CH)
        sid = lax.axis_index("s")

        @pl.when(sid == 0)
        def _stage_table():
            pltpu.sync_copy(table_hbm, tab_sh)

        pltpu.sync_copy(idx_hbm.at[wid], idx_v)
        plsc.subcore_barrier()
        table_hbm = tab_sh

        gathers = [None] * nbuf
        stores = [None] * nbuf
        for c in range(min(nbuf, nch)):
            gathers[c] = pltpu.async_copy(table_hbm.at[idx_v.at[c]], bufs[c], gsems[c])
        for c in range(nch):
            b = c % nbuf
            gathers[b].wait()
            stores[b] = pltpu.async_copy(
                bufs[b], out_hbm.at[pl.ds(base + c * _CH, _CH)], ssems[b]
            )
            nxt = c + nbuf
            if nxt < nch:
                stores[b].wait()
                gathers[b] = pltpu.async_copy(
                    table_hbm.at[idx_v.at[nxt]], bufs[b], gsems[b]
                )
                stores[b] = None
        for s in stores:
            if s is not None:
                s.wait()

    return body(idx3, table)


def kernel(input_ids, attention_mask, embed_weight):
    del attention_mask
    batch, seq = input_ids.shape
    vocab, hidden = embed_weight.shape
    n_rows = batch * seq
    nch = n_rows // (_NW * _CH)
    ids = input_ids.reshape(-1).astype(jnp.int32)
    idx3 = ids.reshape(_NW, nch, _CH)
    table = embed_weight.astype(jnp.float32)
    out = _sc_gather(idx3, table, nch, hidden)
    return out.reshape(batch, seq, hidden)


# speedup vs baseline: 4.4847x; 1.0174x over previous
import functools

import jax
import jax.numpy as jnp
from jax import lax
from jax.experimental import pallas as pl
from jax.experimental.pallas import tpu as pltpu
from jax.experimental.pallas import tpu_sc as plsc


_INFO = plsc.get_sparse_core_info()
_NC = _INFO.num_cores
_NS = _INFO.num_subcores
_NW = _NC * _NS
_CH = 128


@functools.partial(jax.jit, static_argnums=(2, 3))
def _sc_gather(idx_flat, table, nch, hidden):
    n_rows = _NW * nch * _CH
    mesh = plsc.VectorSubcoreMesh(core_axis_name="c", subcore_axis_name="s")

    nbuf = 3
    vocab = table.shape[0]

    @functools.partial(
        pl.kernel,
        out_type=jax.ShapeDtypeStruct((n_rows // _CH, _CH, hidden), jnp.float32),
        mesh=mesh,
        scratch_types=[
            pltpu.VMEM((nch * _CH,), jnp.int32),
            [pltpu.VMEM((1, _CH, hidden), jnp.float32)] * nbuf,
            pltpu.VMEM_SHARED((vocab, hidden), jnp.float32),
            [pltpu.SemaphoreType.DMA] * nbuf,
            [pltpu.SemaphoreType.DMA] * nbuf,
        ],
    )
    def body(idx_hbm, table_hbm, out_hbm, idx_v, bufs, tab_sh, gsems, ssems):
        wid = lax.axis_index("s") * _NC + lax.axis_index("c")
        base = wid * nch
        sid = lax.axis_index("s")

        @pl.when(sid == 0)
        def _stage_table():
            pltpu.sync_copy(table_hbm, tab_sh)

        pltpu.sync_copy(idx_hbm.at[pl.ds(wid * nch * _CH, nch * _CH)], idx_v)
        plsc.subcore_barrier()

        def gather(c, b):
            return pltpu.async_copy(
                tab_sh.at[idx_v.at[pl.ds(c * _CH, _CH)]], bufs[b].at[0], gsems[b]
            )

        gathers = [None] * nbuf
        stores = [None] * nbuf
        for c in range(min(nbuf, nch)):
            gathers[c] = gather(c, c)
        for c in range(nch):
            b = c % nbuf
            gathers[b].wait()
            stores[b] = pltpu.async_copy(
                bufs[b], out_hbm.at[pl.ds(base + c, 1)], ssems[b]
            )
            nxt = c + nbuf
            if nxt < nch:
                stores[b].wait()
                gathers[b] = gather(nxt, b)
                stores[b] = None
        for s in stores:
            if s is not None:
                s.wait()

    return body(idx_flat, table)


def kernel(input_ids, attention_mask, embed_weight):
    del attention_mask
    batch, seq = input_ids.shape
    vocab, hidden = embed_weight.shape
    n_rows = batch * seq
    nch = n_rows // (_NW * _CH)
    ids = input_ids.reshape(-1).astype(jnp.int32)
    table = embed_weight.astype(jnp.float32)
    out = _sc_gather(ids, table, nch, hidden)
    return out.reshape(batch, seq, hidden)
